# baseline (speedup 1.0000x reference)
import functools

import jax
import jax.numpy as jnp
from jax import lax
from jax.experimental import pallas as pl
from jax.experimental.pallas import tpu as pltpu
from jax.experimental.pallas import tpu_sc as plsc

VOCAB = 100000
D1 = 20
D2 = 32
B = 1024
F = 26

D = D1 * D2
NW = 32
DD_PER_W = D // NW
SEG0 = 50048
SEG1 = 49920
TAILW = 128
NBLK = B // 16


def _make_gather():
    mesh = plsc.VectorSubcoreMesh(core_axis_name="c", subcore_axis_name="s")

    @functools.partial(
        pl.kernel,
        mesh=mesh,
        out_type=jax.ShapeDtypeStruct((F, D, B), jnp.float32),
        scratch_types=[
            pltpu.VMEM((SEG0,), jnp.float32),
            pltpu.VMEM((24, TAILW), jnp.float32),
            pltpu.VMEM((F, B), jnp.int32),
            pltpu.VMEM((F, B), jnp.float32),
        ],
        compiler_params=pltpu.CompilerParams(needs_layout_passes=False),
    )
    def gather_kernel(mt_hbm, tail_hbm, idx_hbm, out_hbm, row_v, tail_v, idx_v, obuf):
        wid = lax.axis_index("s") * 2 + lax.axis_index("c")
        dd0 = wid * DD_PER_W
        dd0_al = pl.multiple_of(dd0 - lax.rem(dd0, 8), 8)

        pltpu.sync_copy(idx_hbm, idx_v)
        pltpu.sync_copy(tail_hbm.at[pl.ds(dd0_al, 24)], tail_v)

        def per_dd(k, _):
            dd = dd0 + k
            loc = dd - dd0_al

            pltpu.sync_copy(mt_hbm.at[dd, pl.ds(0, SEG0)], row_v)

            def pass0_f(f, _):
                def pass0_j(j, _):
                    vidx = idx_v[f, pl.ds(j * 16, 16)]
                    msk = vidx < SEG0
                    vals = plsc.load_gather(row_v, [vidx], mask=msk)
                    obuf[f, pl.ds(j * 16, 16)] = vals
                    return 0

                return lax.fori_loop(0, NBLK, pass0_j, 0)

            lax.fori_loop(0, F, pass0_f, 0)

            pltpu.sync_copy(
                mt_hbm.at[dd, pl.ds(SEG0, SEG1)], row_v.at[pl.ds(0, SEG1)]
            )
            row_v[pl.ds(SEG1, 16)] = tail_v[loc, pl.ds(96, 16)]
            row_v[pl.ds(SEG1 + 16, 16)] = tail_v[loc, pl.ds(112, 16)]

            def pass1_f(f, _):
                vf = jnp.full((16,), f, jnp.int32)
                vlane = lax.iota(jnp.int32, 16)

                def pass1_j(j, _):
                    vidx = idx_v[f, pl.ds(j * 16, 16)]
                    msk = vidx >= SEG0
                    vals = plsc.load_gather(row_v, [vidx - SEG0], mask=msk)
                    plsc.store_scatter(obuf, [vf, vlane + j * 16], vals, mask=msk)
                    return 0

                return lax.fori_loop(0, NBLK, pass1_j, 0)

            lax.fori_loop(0, F, pass1_f, 0)

            pltpu.sync_copy(obuf, out_hbm.at[:, dd])
            return 0

        lax.fori_loop(0, DD_PER_W, per_dd, 0)

    return gather_kernel


_gather = _make_gather()


def kernel(i, m):
    mt = jnp.transpose(m, (1, 2, 0)).reshape(D, VOCAB)
    mtail = jnp.transpose(m[VOCAB - TAILW :], (1, 2, 0)).reshape(D, TAILW)
    iT = jnp.transpose(i.astype(jnp.int32), (1, 0))
    out_t = _gather(mt, mtail, iT)
    return jnp.transpose(out_t.reshape(F, D1, D2, B), (3, 0, 1, 2))


# speedup vs baseline: 1.4684x; 1.4684x over previous
import functools

import jax
import jax.numpy as jnp
from jax import lax
from jax.experimental import pallas as pl
from jax.experimental.pallas import tpu as pltpu
from jax.experimental.pallas import tpu_sc as plsc

VOCAB = 100000
D1 = 20
D2 = 32
B = 1024
F = 26

D = D1 * D2
NW = 32
DD_PER_W = D // NW
SEG = 99968
TAILW = 128
NBLK = B // 16


def _make_gather():
    mesh = plsc.VectorSubcoreMesh(core_axis_name="c", subcore_axis_name="s")

    @functools.partial(
        pl.kernel,
        mesh=mesh,
        out_type=jax.ShapeDtypeStruct((F, D, B), jnp.float32),
        scratch_types=[
            pltpu.VMEM((VOCAB,), jnp.float32),
            pltpu.VMEM((24, TAILW), jnp.float32),
            pltpu.VMEM((2, B), jnp.int32),
            pltpu.VMEM((4, B), jnp.float32),
            pltpu.SemaphoreType.DMA,
            pltpu.SemaphoreType.DMA,
        ],
        compiler_params=pltpu.CompilerParams(needs_layout_passes=False),
    )
    def gather_kernel(
        mt_hbm, tail_hbm, idx_hbm, out_hbm, row_v, tail_v, idxb, obuf, isem, osem
    ):
        wid = lax.axis_index("s") * 2 + lax.axis_index("c")
        dd0 = wid * DD_PER_W
        dd0_al = pl.multiple_of(dd0 - lax.rem(dd0, 8), 8)

        pltpu.sync_copy(tail_hbm.at[pl.ds(dd0_al, 24)], tail_v)

        def per_dd(k, _):
            dd = dd0 + k
            loc = dd - dd0_al

            pltpu.async_copy(idx_hbm.at[0], idxb.at[0], isem)

            pltpu.sync_copy(mt_hbm.at[dd, pl.ds(0, SEG)], row_v.at[pl.ds(0, SEG)])
            row_v[pl.ds(SEG, 16)] = tail_v[loc, pl.ds(96, 16)]
            row_v[pl.ds(SEG + 16, 16)] = tail_v[loc, pl.ds(112, 16)]

            def per_f(f, _):
                fb = lax.rem(f, 2)
                ob = lax.rem(f, 4)
                g = k * F + f

                pltpu.make_async_copy(idx_hbm.at[f], idxb.at[fb], isem).wait()

                @pl.when(f < F - 1)
                def _():
                    pltpu.async_copy(idx_hbm.at[f + 1], idxb.at[1 - fb], isem)

                @pl.when(g >= 4)
                def _():
                    pltpu.make_async_copy(
                        obuf.at[ob], out_hbm.at[f, dd], osem
                    ).wait()

                def swp(j, _):
                    vidx = idxb[fb, pl.ds(j * 16, 16)]
                    obuf[ob, pl.ds(j * 16, 16)] = plsc.load_gather(row_v, [vidx])
                    return 0

                lax.fori_loop(0, NBLK, swp, 0)

                pltpu.async_copy(obuf.at[ob], out_hbm.at[f, dd], osem)
                return 0

            lax.fori_loop(0, F, per_f, 0)
            return 0

        lax.fori_loop(0, DD_PER_W, per_dd, 0)

        for t in range(4):
            pltpu.make_async_copy(obuf.at[t], out_hbm.at[t, dd0], osem).wait()

    return gather_kernel


_gather = _make_gather()


def kernel(i, m):
    mt = jnp.transpose(m, (1, 2, 0)).reshape(D, VOCAB)
    mtail = jnp.transpose(m[VOCAB - TAILW :], (1, 2, 0)).reshape(D, TAILW)
    iT = jnp.transpose(i.astype(jnp.int32), (1, 0))
    out_t = _gather(mt, mtail, iT)
    return jnp.transpose(out_t.reshape(F, D1, D2, B), (3, 0, 1, 2))


# speedup vs baseline: 1.5218x; 1.0364x over previous
import functools

import jax
import jax.numpy as jnp
from jax import lax
from jax.experimental import pallas as pl
from jax.experimental.pallas import tpu as pltpu
from jax.experimental.pallas import tpu_sc as plsc

VOCAB = 100000
D1 = 20
D2 = 32
B = 1024
F = 26

D = D1 * D2
NW = 32
DD_PER_W = D // NW
SEG = 99968
TAILW = 128
NBLK = B // 16


def _make_gather():
    mesh = plsc.VectorSubcoreMesh(core_axis_name="c", subcore_axis_name="s")

    @functools.partial(
        pl.kernel,
        mesh=mesh,
        out_type=jax.ShapeDtypeStruct((F, D, B), jnp.float32),
        scratch_types=[
            pltpu.VMEM((VOCAB,), jnp.float32),
            pltpu.VMEM((24, TAILW), jnp.float32),
            pltpu.VMEM((2, B), jnp.int32),
            pltpu.VMEM((4, B), jnp.float32),
            pltpu.SemaphoreType.DMA,
            pltpu.SemaphoreType.DMA,
            pltpu.SemaphoreType.DMA,
        ],
        compiler_params=pltpu.CompilerParams(needs_layout_passes=False),
    )
    def gather_kernel(
        mt_hbm, tail_hbm, idx_hbm, out_hbm, row_v, tail_v, idxb, obuf,
        isem, osem, rsem,
    ):
        wid = lax.axis_index("s") * 2 + lax.axis_index("c")
        dd0 = wid * DD_PER_W
        dd0_al = pl.multiple_of(dd0 - lax.rem(dd0, 8), 8)

        pltpu.sync_copy(tail_hbm.at[pl.ds(dd0_al, 24)], tail_v)

        def per_dd(k, _):
            dd = dd0 + k
            loc = dd - dd0_al

            pltpu.async_copy(idx_hbm.at[0], idxb.at[0], isem)

            for o, l in ((0, 25088), (25088, 25088), (50176, 25088), (75264, 24704)):
                pltpu.async_copy(
                    mt_hbm.at[dd, pl.ds(o, l)], row_v.at[pl.ds(o, l)], rsem
                )
            for o, l in ((0, 25088), (25088, 25088), (50176, 25088), (75264, 24704)):
                pltpu.make_async_copy(
                    mt_hbm.at[dd, pl.ds(o, l)], row_v.at[pl.ds(o, l)], rsem
                ).wait()
            row_v[pl.ds(SEG, 16)] = tail_v[loc, pl.ds(96, 16)]
            row_v[pl.ds(SEG + 16, 16)] = tail_v[loc, pl.ds(112, 16)]

            def per_f(f, _):
                fb = lax.rem(f, 2)
                ob = lax.rem(f, 4)
                g = k * F + f

                pltpu.make_async_copy(idx_hbm.at[f], idxb.at[fb], isem).wait()

                @pl.when(f < F - 1)
                def _():
                    pltpu.async_copy(idx_hbm.at[f + 1], idxb.at[1 - fb], isem)

                @pl.when(g >= 4)
                def _():
                    pltpu.make_async_copy(
                        obuf.at[ob], out_hbm.at[f, dd], osem
                    ).wait()

                def swp(j, _):
                    base = j * 64
                    v0 = idxb[fb, pl.ds(base, 16)]
                    v1 = idxb[fb, pl.ds(base + 16, 16)]
                    v2 = idxb[fb, pl.ds(base + 32, 16)]
                    v3 = idxb[fb, pl.ds(base + 48, 16)]
                    g0 = plsc.load_gather(row_v, [v0])
                    g1 = plsc.load_gather(row_v, [v1])
                    g2 = plsc.load_gather(row_v, [v2])
                    g3 = plsc.load_gather(row_v, [v3])
                    obuf[ob, pl.ds(base, 16)] = g0
                    obuf[ob, pl.ds(base + 16, 16)] = g1
                    obuf[ob, pl.ds(base + 32, 16)] = g2
                    obuf[ob, pl.ds(base + 48, 16)] = g3
                    return 0

                lax.fori_loop(0, NBLK // 4, swp, 0)

                pltpu.async_copy(obuf.at[ob], out_hbm.at[f, dd], osem)
                return 0

            lax.fori_loop(0, F, per_f, 0)
            return 0

        lax.fori_loop(0, DD_PER_W, per_dd, 0)

        for t in range(4):
            pltpu.make_async_copy(obuf.at[t], out_hbm.at[t, dd0], osem).wait()

    return gather_kernel


_gather = _make_gather()


def kernel(i, m):
    mt = jnp.transpose(m, (1, 2, 0)).reshape(D, VOCAB)
    mtail = jnp.transpose(m[VOCAB - TAILW :], (1, 2, 0)).reshape(D, TAILW)
    iT = jnp.transpose(i.astype(jnp.int32), (1, 0))
    out_t = _gather(mt, mtail, iT)
    return jnp.transpose(out_t.reshape(F, D1, D2, B), (3, 0, 1, 2))


# speedup vs baseline: 2.6746x; 1.7575x over previous
import functools

import jax
import jax.numpy as jnp
from jax import lax
from jax.experimental import pallas as pl
from jax.experimental.pallas import tpu as pltpu
from jax.experimental.pallas import tpu_sc as plsc

VOCAB = 100000
D1 = 20
D2 = 32
B = 1024
F = 26

D = D1 * D2
NW = 32
DD_PER_W = D // NW
SEG = 99968
TAILW = 128
NBLK = B // 16

QUARTERS = ((0, 25088), (25088, 25088), (50176, 25088), (75264, 24704))


def _make_gather():
    mesh = plsc.VectorSubcoreMesh(core_axis_name="c", subcore_axis_name="s")

    @functools.partial(
        pl.kernel,
        mesh=mesh,
        out_type=jax.ShapeDtypeStruct((F, D, B), jnp.float32),
        scratch_types=[
            pltpu.VMEM((VOCAB,), jnp.float32),
            pltpu.VMEM((TAILW,), jnp.float32),
            pltpu.VMEM((4, B), jnp.int32),
            pltpu.VMEM((4, B), jnp.float32),
            pltpu.SemaphoreType.DMA,
            pltpu.SemaphoreType.DMA,
            pltpu.SemaphoreType.DMA,
        ],
        compiler_params=pltpu.CompilerParams(needs_layout_passes=False),
    )
    def gather_kernel(
        mt_hbm, tail_hbm, idx_hbm, out_hbm, row_v, tailrow_v, idxb, obuf,
        isem, osem, rsem,
    ):
        wid = lax.axis_index("s") * 2 + lax.axis_index("c")
        dd0 = wid * DD_PER_W

        def per_dd(k, _):
            dd = dd0 + k

            for ff in range(3):
                pltpu.async_copy(idx_hbm.at[ff], idxb.at[ff], isem)

            for o, l in QUARTERS:
                pltpu.async_copy(
                    mt_hbm.at[dd, pl.ds(o, l)], row_v.at[pl.ds(o, l)], rsem
                )
            pltpu.async_copy(tail_hbm.at[dd], tailrow_v, rsem)
            for o, l in QUARTERS:
                pltpu.make_async_copy(
                    mt_hbm.at[dd, pl.ds(o, l)], row_v.at[pl.ds(o, l)], rsem
                ).wait()
            pltpu.make_async_copy(tail_hbm.at[dd], tailrow_v, rsem).wait()
            row_v[pl.ds(SEG, 16)] = tailrow_v[pl.ds(96, 16)]
            row_v[pl.ds(SEG + 16, 16)] = tailrow_v[pl.ds(112, 16)]

            def per_f(f, _):
                fb = lax.rem(f, 4)
                ob = lax.rem(f, 4)
                g = k * F + f

                pltpu.make_async_copy(idx_hbm.at[f], idxb.at[fb], isem).wait()

                @pl.when(f < F - 3)
                def _():
                    pltpu.async_copy(
                        idx_hbm.at[f + 3], idxb.at[lax.rem(f + 3, 4)], isem
                    )

                @pl.when(g >= 4)
                def _():
                    pltpu.make_async_copy(
                        obuf.at[ob], out_hbm.at[f, dd], osem
                    ).wait()

                def swp(j, _):
                    base = j * 64
                    v0 = idxb[fb, pl.ds(base, 16)]
                    v1 = idxb[fb, pl.ds(base + 16, 16)]
                    v2 = idxb[fb, pl.ds(base + 32, 16)]
                    v3 = idxb[fb, pl.ds(base + 48, 16)]
                    g0 = plsc.load_gather(row_v, [v0])
                    g1 = plsc.load_gather(row_v, [v1])
                    g2 = plsc.load_gather(row_v, [v2])
                    g3 = plsc.load_gather(row_v, [v3])
                    obuf[ob, pl.ds(base, 16)] = g0
                    obuf[ob, pl.ds(base + 16, 16)] = g1
                    obuf[ob, pl.ds(base + 32, 16)] = g2
                    obuf[ob, pl.ds(base + 48, 16)] = g3
                    return 0

                lax.fori_loop(0, NBLK // 4, swp, 0)

                pltpu.async_copy(obuf.at[ob], out_hbm.at[f, dd], osem)
                return 0

            lax.fori_loop(0, F, per_f, 0)
            return 0

        lax.fori_loop(0, DD_PER_W, per_dd, 0)

        for t in range(4):
            pltpu.make_async_copy(obuf.at[t], out_hbm.at[t, dd0], osem).wait()

    return gather_kernel


_gather = _make_gather()


def kernel(i, m):
    mt = jnp.transpose(m, (1, 2, 0)).reshape(D, VOCAB)
    mtail = jnp.transpose(m[VOCAB - TAILW :], (1, 2, 0)).reshape(D, TAILW)
    iT = jnp.transpose(i.astype(jnp.int32), (1, 0))
    out_t = _gather(mt, mtail, iT)
    return jnp.transpose(out_t.reshape(F, D1, D2, B), (3, 0, 1, 2))
